# Initial kernel scaffold; baseline (speedup 1.0000x reference)
import jax
import jax.numpy as jnp
from jax.experimental import pallas as pl


def kernel(x, seq_embed, W_gat, att_src, att_dst, b_gat, W_gcn, b_gcn, W_fcg1, b_fcg1, W_fcg2, b_fcg2, Wc1, bc1, Wc2, bc2, Wc3, bc3, Wc4, bc4, W_fcxt, b_fcxt, W_fc1, b_fc1, W_fc2, b_fc2, W_out, b_out, edge_index, batch):
    raise NotImplementedError("write your pallas kernel here")


# speedup vs baseline: 1.0483x; 1.0483x over previous
import functools

import jax
import jax.numpy as jnp
from jax import lax
from jax.experimental import pallas as pl
from jax.experimental.pallas import tpu as pltpu

N = 10000
E = 160000
D = 78
H = 10
B = 4
L = 1020
SD = 1024

NP = 10240
BN = 256
F = 784
EP = 170496

_INTERPRET = False


def _pc(body, grid, in_specs, out_specs, out_shape, **kw):
    return pl.pallas_call(
        body, grid=grid, in_specs=in_specs, out_specs=out_specs,
        out_shape=out_shape, interpret=_INTERPRET, **kw)


def _k1_body(x_ref, w_ref, as_ref, ad_ref, h_ref, asq_ref, adq_ref):
    h = jnp.dot(x_ref[...], w_ref[...], preferred_element_type=jnp.float32)
    h_ref[...] = h
    asq_ref[...] = jnp.dot(h, as_ref[...], preferred_element_type=jnp.float32)
    adq_ref[...] = jnp.dot(h, ad_ref[...], preferred_element_type=jnp.float32)


def _k1(x_p, w_p, a_s, a_d):
    return _pc(
        _k1_body, grid=(NP // BN,),
        in_specs=[
            pl.BlockSpec((BN, 128), lambda i: (i, 0)),
            pl.BlockSpec((128, F), lambda i: (0, 0)),
            pl.BlockSpec((F, 16), lambda i: (0, 0)),
            pl.BlockSpec((F, 16), lambda i: (0, 0)),
        ],
        out_specs=[
            pl.BlockSpec((BN, F), lambda i: (i, 0)),
            pl.BlockSpec((BN, 16), lambda i: (i, 0)),
            pl.BlockSpec((BN, 16), lambda i: (i, 0)),
        ],
        out_shape=[
            jax.ShapeDtypeStruct((NP, F), jnp.float32),
            jax.ShapeDtypeStruct((NP, 16), jnp.float32),
            jax.ShapeDtypeStruct((NP, 16), jnp.float32),
        ],
    )(x_p, w_p, a_s, a_d)


def _k2_body(g_ref, b_ref, w_ref, o_ref):
    g = jnp.maximum(g_ref[...] + b_ref[...], 0.0)
    o_ref[...] = jnp.dot(g, w_ref[...], preferred_element_type=jnp.float32)


def _k2(gat_out, b_gat_p, w_gcn_p):
    return _pc(
        _k2_body, grid=(NP // BN,),
        in_specs=[
            pl.BlockSpec((BN, F), lambda i: (i, 0)),
            pl.BlockSpec((1, F), lambda i: (0, 0)),
            pl.BlockSpec((F, F), lambda i: (0, 0)),
        ],
        out_specs=pl.BlockSpec((BN, F), lambda i: (i, 0)),
        out_shape=jax.ShapeDtypeStruct((NP, F), jnp.float32),
    )(gat_out, b_gat_p, w_gcn_p)


def _k3_body(g_ref, b_ref, oh_ref, gm_ref, gs_ref):
    i = pl.program_id(0)

    @pl.when(i == 0)
    def _():
        gm_ref[...] = jnp.full_like(gm_ref, -1e30)
        gs_ref[...] = jnp.zeros_like(gs_ref)

    g = jnp.maximum(g_ref[...] + b_ref[...], 0.0)
    oh = oh_ref[...]
    for q in range(B):
        col = oh[:, q:q + 1]
        m = g * col + (col - 1.0) * 1e30
        gm_ref[q:q + 1, :] = jnp.maximum(
            gm_ref[q:q + 1, :], jnp.max(m, axis=0, keepdims=True))
        gs_ref[q:q + 1, :] = gs_ref[q:q + 1, :] + jnp.sum(
            g * col, axis=0, keepdims=True)


def _k3(gcn_out, b_gcn_p, onehot):
    return _pc(
        _k3_body, grid=(NP // BN,),
        in_specs=[
            pl.BlockSpec((BN, F), lambda i: (i, 0)),
            pl.BlockSpec((1, F), lambda i: (0, 0)),
            pl.BlockSpec((BN, 8), lambda i: (i, 0)),
        ],
        out_specs=[
            pl.BlockSpec((8, F), lambda i: (0, 0)),
            pl.BlockSpec((8, F), lambda i: (0, 0)),
        ],
        out_shape=[
            jax.ShapeDtypeStruct((8, F), jnp.float32),
            jax.ShapeDtypeStruct((8, F), jnp.float32),
        ],
    )(gcn_out, b_gcn_p, onehot)


def _k4_body(gm_ref, ga_ref, w1a_ref, w1b_ref, b1_ref, w2_ref, b2_ref, o_ref):
    y = jnp.dot(gm_ref[...], w1a_ref[...], preferred_element_type=jnp.float32)
    y = y + jnp.dot(ga_ref[...], w1b_ref[...], preferred_element_type=jnp.float32)
    y = jnp.maximum(y + b1_ref[...], 0.0)
    o_ref[...] = jnp.dot(y, w2_ref[...], preferred_element_type=jnp.float32) + b2_ref[...]


def _k4(gm, ga, w1a, w1b, b1, w2, b2):
    G1 = 1536
    return _pc(
        _k4_body, grid=(1,),
        in_specs=[
            pl.BlockSpec((8, F), lambda i: (0, 0)),
            pl.BlockSpec((8, F), lambda i: (0, 0)),
            pl.BlockSpec((F, G1), lambda i: (0, 0)),
            pl.BlockSpec((F, G1), lambda i: (0, 0)),
            pl.BlockSpec((1, G1), lambda i: (0, 0)),
            pl.BlockSpec((G1, 128), lambda i: (0, 0)),
            pl.BlockSpec((1, 128), lambda i: (0, 0)),
        ],
        out_specs=pl.BlockSpec((8, 128), lambda i: (0, 0)),
        out_shape=jax.ShapeDtypeStruct((8, 128), jnp.float32),
    )(gm, ga, w1a, w1b, b1, w2, b2)


def _conv_body(x_ref, w_ref, b_ref, o_ref, *, k, lb, cin, first):
    i = pl.program_id(1)
    xs_big = x_ref[0, pl.ds(i * lb, lb + 8), :]
    if not first:
        xs_big = jnp.maximum(jnp.maximum(xs_big[:, :cin], xs_big[:, cin:]), 0.0)
    acc = jnp.broadcast_to(b_ref[...], o_ref.shape[1:]).astype(jnp.float32)
    for kk in range(k):
        acc = acc + jnp.dot(xs_big[kk:kk + lb], w_ref[kk],
                            preferred_element_type=jnp.float32)
    o_ref[0] = acc


def _conv_stage(x, w, bias, k, nb, lb, cin, cout, first):
    lout = nb * lb
    return _pc(
        functools.partial(_conv_body, k=k, lb=lb, cin=cin, first=first),
        grid=(B, nb),
        in_specs=[
            pl.BlockSpec((1,) + x.shape[1:], lambda b, i: (b, 0, 0)),
            pl.BlockSpec(w.shape, lambda b, i: (0, 0, 0)),
            pl.BlockSpec((1, cout), lambda b, i: (0, 0)),
        ],
        out_specs=pl.BlockSpec((1, lb, cout), lambda b, i: (b, i, 0)),
        out_shape=jax.ShapeDtypeStruct((B, lout, cout), jnp.float32),
    )(x, w, bias.reshape(1, cout))


def _s5_body(x_ref, o_ref):
    xx = x_ref[0]
    o_ref[0] = jnp.maximum(jnp.maximum(xx[:, :128], xx[:, 128:]), 0.0)


def _s5(x):
    return _pc(
        _s5_body, grid=(B,),
        in_specs=[pl.BlockSpec((1, 64, 256), lambda b: (b, 0, 0))],
        out_specs=pl.BlockSpec((1, 64, 128), lambda b: (b, 0, 0)),
        out_shape=jax.ShapeDtypeStruct((B, 64, 128), jnp.float32),
    )(x)


def _s6_body(xt_ref, wp_ref, bp_ref, gf_ref, w1a_ref, w1b_ref, b1_ref,
             w2_ref, b2_ref, wo_ref, bo_ref, o_ref, acc_ref):
    j = pl.program_id(0)

    @pl.when(j == 0)
    def _():
        acc_ref[...] = jnp.zeros_like(acc_ref)

    acc_ref[...] += jnp.dot(xt_ref[...], wp_ref[...],
                            preferred_element_type=jnp.float32)

    @pl.when(j == pl.num_programs(0) - 1)
    def _():
        xt = jnp.maximum(acc_ref[...] + bp_ref[...], 0.0)
        y = jnp.dot(gf_ref[...], w1a_ref[...], preferred_element_type=jnp.float32)
        y = y + jnp.dot(xt, w1b_ref[...], preferred_element_type=jnp.float32)
        y = jnp.maximum(y + b1_ref[...], 0.0)
        y = jnp.maximum(jnp.dot(y, w2_ref[...], preferred_element_type=jnp.float32)
                        + b2_ref[...], 0.0)
        o_ref[...] = jnp.dot(y, wo_ref[...], preferred_element_type=jnp.float32) + bo_ref[...]


def _s6(xt_flat, wp, bp, gf, w1a, w1b, b1, w2, b2, wo, bo):
    KB = 2048
    return _pc(
        _s6_body, grid=(8192 // KB,),
        in_specs=[
            pl.BlockSpec((8, KB), lambda j: (0, j)),
            pl.BlockSpec((KB, 1024), lambda j: (j, 0)),
            pl.BlockSpec((1, 1024), lambda j: (0, 0)),
            pl.BlockSpec((8, 128), lambda j: (0, 0)),
            pl.BlockSpec((128, 1024), lambda j: (0, 0)),
            pl.BlockSpec((1024, 1024), lambda j: (0, 0)),
            pl.BlockSpec((1, 1024), lambda j: (0, 0)),
            pl.BlockSpec((1024, 256), lambda j: (0, 0)),
            pl.BlockSpec((1, 256), lambda j: (0, 0)),
            pl.BlockSpec((256, 128), lambda j: (0, 0)),
            pl.BlockSpec((1, 128), lambda j: (0, 0)),
        ],
        out_specs=pl.BlockSpec((8, 128), lambda j: (0, 0)),
        out_shape=jax.ShapeDtypeStruct((8, 128), jnp.float32),
        scratch_shapes=[pltpu.VMEM((8, 1024), jnp.float32)],
    )(xt_flat, wp, bp, gf, w1a, w1b, b1, w2, b2, wo, bo)


def kernel(x, seq_embed, W_gat, att_src, att_dst, b_gat, W_gcn, b_gcn,
           W_fcg1, b_fcg1, W_fcg2, b_fcg2, Wc1, bc1, Wc2, bc2, Wc3, bc3,
           Wc4, bc4, W_fcxt, b_fcxt, W_fc1, b_fc1, W_fc2, b_fc2,
           W_out, b_out, edge_index, batch):
    f32 = jnp.float32

    x_p = jnp.zeros((NP, 128), f32).at[:N, :D].set(x)
    w_gat_p = jnp.zeros((128, F), f32).at[:D, :H * D].set(W_gat)
    rows = jnp.arange(H * D)
    a_s = jnp.zeros((F, 16), f32).at[rows, rows // D].set(att_src.reshape(-1))
    a_d = jnp.zeros((F, 16), f32).at[rows, rows // D].set(att_dst.reshape(-1))
    b_gat_p = jnp.zeros((1, F), f32).at[0, :H * D].set(b_gat)
    w_gcn_p = jnp.zeros((F, F), f32).at[:H * D, :H * D].set(W_gcn)
    b_gcn_p = jnp.zeros((1, F), f32).at[0, :H * D].set(b_gcn)

    src = jnp.concatenate([edge_index[0], jnp.arange(N, dtype=edge_index.dtype)])
    dst = jnp.concatenate([edge_index[1], jnp.arange(N, dtype=edge_index.dtype)])

    h, as_q, ad_q = _k1(x_p, w_gat_p, a_s, a_d)

    as_n = as_q[:N, :H]
    ad_n = ad_q[:N, :H]
    e = jax.nn.leaky_relu(as_n[src] + ad_n[dst], 0.2)
    p = jnp.exp(e)
    z = jax.ops.segment_sum(p, dst, num_segments=N)
    alpha = p / (z[dst] + 1e-16)
    h3 = h[:N, :H * D].reshape(N, H, D)
    gat = jax.ops.segment_sum(h3[src] * alpha[:, :, None], dst, num_segments=N)
    gat_out = jnp.zeros((NP, F), f32).at[:N, :H * D].set(gat.reshape(N, H * D))

    h2 = _k2(gat_out, b_gat_p, w_gcn_p)

    deg = jax.ops.segment_sum(jnp.ones_like(src, dtype=f32), dst, num_segments=N)
    dis = jnp.where(deg > 0, lax.rsqrt(deg), 0.0)
    norm = dis[src] * dis[dst]
    h2n = h2[:N, :H * D]
    gcn = jax.ops.segment_sum(h2n[src] * norm[:, None], dst, num_segments=N)
    gcn_out = jnp.zeros((NP, F), f32).at[:N, :H * D].set(gcn)

    onehot = jnp.zeros((NP, 8), f32).at[jnp.arange(N), batch].set(1.0)
    gm, gs = _k3(gcn_out, b_gcn_p, onehot)
    cnt = jnp.maximum(jnp.sum(onehot, axis=0), 1.0)
    ga = gs / cnt[:, None]
    G1 = 1536
    w1a = jnp.zeros((F, G1), f32).at[:H * D, :1500].set(W_fcg1[:H * D])
    w1b = jnp.zeros((F, G1), f32).at[:H * D, :1500].set(W_fcg1[H * D:])
    b1 = jnp.zeros((1, G1), f32).at[0, :1500].set(b_fcg1)
    w2 = jnp.zeros((G1, 128), f32).at[:1500].set(W_fcg2)
    gfeat = _k4(gm, ga, w1a, w1b, b1, w2, b_fcg2.reshape(1, 128))

    xs_p = jnp.zeros((B, 1032, SD), f32).at[:, :L, :].set(seq_embed)
    y1 = _conv_stage(xs_p, jnp.transpose(Wc1, (2, 1, 0)), bc1, 5, 4, 256, SD, 256, True)
    x2 = jnp.zeros((B, 520, 512), f32).at[:, :512, :].set(y1.reshape(B, 512, 512))
    y2 = _conv_stage(x2, jnp.transpose(Wc2, (2, 1, 0)), bc2, 5, 2, 256, 256, SD, False)
    x3 = jnp.zeros((B, 264, 2048), f32).at[:, :256, :].set(y2.reshape(B, 256, 2048))
    y3 = _conv_stage(x3, jnp.transpose(Wc3, (2, 1, 0)), bc3, 5, 1, 256, SD, 256, False)
    x4 = jnp.zeros((B, 136, 512), f32).at[:, :128, :].set(y3.reshape(B, 128, 512))
    y4 = _conv_stage(x4, jnp.transpose(Wc4, (2, 1, 0)), bc4, 3, 1, 128, 256, 128, False)
    pooled4 = _s5(y4.reshape(B, 64, 256))
    xt_flat = jnp.zeros((8, 8192), f32).at[:B].set(pooled4.reshape(B, 8192))

    lidx = jnp.arange(8192) // 128
    cidx = jnp.arange(8192) % 128
    srcrow = jnp.where(lidx < 61, cidx * 61 + lidx, 0)
    wp = jnp.where((lidx < 61)[:, None], W_fcxt[srcrow], 0.0)
    gf = jnp.zeros((8, 128), f32).at[:B].set(gfeat[:B])
    w1a_f = W_fc1[:128]
    w1b_f = W_fc1[128:]
    wo = jnp.zeros((256, 128), f32).at[:, 0].set(W_out[:, 0])
    bo = jnp.zeros((1, 128), f32).at[0, 0].set(b_out[0])
    out = _s6(xt_flat, wp, b_fcxt.reshape(1, 1024), gf, w1a_f, w1b_f,
              b_fc1.reshape(1, 1024), W_fc2, b_fc2.reshape(1, 256),
              wo, bo)
    return out[:B, :1]


# speedup vs baseline: 3.3382x; 3.1843x over previous
import functools

import jax
import jax.numpy as jnp
from jax import lax
from jax.experimental import pallas as pl
from jax.experimental.pallas import tpu as pltpu
from jax.experimental.pallas import tpu_sc as plsc

N = 10000
E = 160000
D = 78
H = 10
B = 4
L = 1020
SD = 1024

NP = 10240
BN = 256
F = 784
EP = 170496

_INTERPRET = False


def _pc(body, grid, in_specs, out_specs, out_shape, **kw):
    return pl.pallas_call(
        body, grid=grid, in_specs=in_specs, out_specs=out_specs,
        out_shape=out_shape, interpret=_INTERPRET, **kw)


def _k1_body(x_ref, w_ref, as_ref, ad_ref, h_ref, asq_ref, adq_ref):
    h = jnp.dot(x_ref[...], w_ref[...], preferred_element_type=jnp.float32)
    h_ref[...] = h
    asq_ref[...] = jnp.dot(h, as_ref[...], preferred_element_type=jnp.float32)
    adq_ref[...] = jnp.dot(h, ad_ref[...], preferred_element_type=jnp.float32)


def _k1(x_p, w_p, a_s, a_d):
    return _pc(
        _k1_body, grid=(NP // BN,),
        in_specs=[
            pl.BlockSpec((BN, 128), lambda i: (i, 0)),
            pl.BlockSpec((128, F), lambda i: (0, 0)),
            pl.BlockSpec((F, 16), lambda i: (0, 0)),
            pl.BlockSpec((F, 16), lambda i: (0, 0)),
        ],
        out_specs=[
            pl.BlockSpec((BN, F), lambda i: (i, 0)),
            pl.BlockSpec((BN, 16), lambda i: (i, 0)),
            pl.BlockSpec((BN, 16), lambda i: (i, 0)),
        ],
        out_shape=[
            jax.ShapeDtypeStruct((NP, F), jnp.float32),
            jax.ShapeDtypeStruct((NP, 16), jnp.float32),
            jax.ShapeDtypeStruct((NP, 16), jnp.float32),
        ],
    )(x_p, w_p, a_s, a_d)


def _k2_body(g_ref, b_ref, w_ref, o_ref):
    g = jnp.maximum(g_ref[...] + b_ref[...], 0.0)
    o_ref[...] = jnp.dot(g, w_ref[...], preferred_element_type=jnp.float32)


def _k2(gat_out, b_gat_p, w_gcn_p):
    return _pc(
        _k2_body, grid=(NP // BN,),
        in_specs=[
            pl.BlockSpec((BN, F), lambda i: (i, 0)),
            pl.BlockSpec((1, F), lambda i: (0, 0)),
            pl.BlockSpec((F, F), lambda i: (0, 0)),
        ],
        out_specs=pl.BlockSpec((BN, F), lambda i: (i, 0)),
        out_shape=jax.ShapeDtypeStruct((NP, F), jnp.float32),
    )(gat_out, b_gat_p, w_gcn_p)


def _k3_body(g_ref, b_ref, oh_ref, gm_ref, gs_ref):
    i = pl.program_id(0)

    @pl.when(i == 0)
    def _():
        gm_ref[...] = jnp.full_like(gm_ref, -1e30)
        gs_ref[...] = jnp.zeros_like(gs_ref)

    g = jnp.maximum(g_ref[...] + b_ref[...], 0.0)
    oh = oh_ref[...]
    for q in range(B):
        col = oh[:, q:q + 1]
        m = g * col + (col - 1.0) * 1e30
        gm_ref[q:q + 1, :] = jnp.maximum(
            gm_ref[q:q + 1, :], jnp.max(m, axis=0, keepdims=True))
        gs_ref[q:q + 1, :] = gs_ref[q:q + 1, :] + jnp.sum(
            g * col, axis=0, keepdims=True)


def _k3(gcn_out, b_gcn_p, onehot):
    return _pc(
        _k3_body, grid=(NP // BN,),
        in_specs=[
            pl.BlockSpec((BN, F), lambda i: (i, 0)),
            pl.BlockSpec((1, F), lambda i: (0, 0)),
            pl.BlockSpec((BN, 8), lambda i: (i, 0)),
        ],
        out_specs=[
            pl.BlockSpec((8, F), lambda i: (0, 0)),
            pl.BlockSpec((8, F), lambda i: (0, 0)),
        ],
        out_shape=[
            jax.ShapeDtypeStruct((8, F), jnp.float32),
            jax.ShapeDtypeStruct((8, F), jnp.float32),
        ],
    )(gcn_out, b_gcn_p, onehot)


def _k4_body(gm_ref, ga_ref, w1a_ref, w1b_ref, b1_ref, w2_ref, b2_ref, o_ref):
    y = jnp.dot(gm_ref[...], w1a_ref[...], preferred_element_type=jnp.float32)
    y = y + jnp.dot(ga_ref[...], w1b_ref[...], preferred_element_type=jnp.float32)
    y = jnp.maximum(y + b1_ref[...], 0.0)
    o_ref[...] = jnp.dot(y, w2_ref[...], preferred_element_type=jnp.float32) + b2_ref[...]


def _k4(gm, ga, w1a, w1b, b1, w2, b2):
    G1 = 1536
    return _pc(
        _k4_body, grid=(1,),
        in_specs=[
            pl.BlockSpec((8, F), lambda i: (0, 0)),
            pl.BlockSpec((8, F), lambda i: (0, 0)),
            pl.BlockSpec((F, G1), lambda i: (0, 0)),
            pl.BlockSpec((F, G1), lambda i: (0, 0)),
            pl.BlockSpec((1, G1), lambda i: (0, 0)),
            pl.BlockSpec((G1, 128), lambda i: (0, 0)),
            pl.BlockSpec((1, 128), lambda i: (0, 0)),
        ],
        out_specs=pl.BlockSpec((8, 128), lambda i: (0, 0)),
        out_shape=jax.ShapeDtypeStruct((8, 128), jnp.float32),
    )(gm, ga, w1a, w1b, b1, w2, b2)


def _conv_body(x_ref, w_ref, b_ref, o_ref, *, k, lb, cin, first):
    i = pl.program_id(1)
    xs_big = x_ref[0, pl.ds(i * lb, lb + 8), :]
    if not first:
        xs_big = jnp.maximum(jnp.maximum(xs_big[:, :cin], xs_big[:, cin:]), 0.0)
    acc = jnp.broadcast_to(b_ref[...], o_ref.shape[1:]).astype(jnp.float32)
    for kk in range(k):
        acc = acc + jnp.dot(xs_big[kk:kk + lb], w_ref[kk],
                            preferred_element_type=jnp.float32)
    o_ref[0] = acc


def _conv_stage(x, w, bias, k, nb, lb, cin, cout, first):
    lout = nb * lb
    return _pc(
        functools.partial(_conv_body, k=k, lb=lb, cin=cin, first=first),
        grid=(B, nb),
        in_specs=[
            pl.BlockSpec((1,) + x.shape[1:], lambda b, i: (b, 0, 0)),
            pl.BlockSpec(w.shape, lambda b, i: (0, 0, 0)),
            pl.BlockSpec((1, cout), lambda b, i: (0, 0)),
        ],
        out_specs=pl.BlockSpec((1, lb, cout), lambda b, i: (b, i, 0)),
        out_shape=jax.ShapeDtypeStruct((B, lout, cout), jnp.float32),
    )(x, w, bias.reshape(1, cout))


def _s5_body(x_ref, o_ref):
    xx = x_ref[0]
    o_ref[0] = jnp.maximum(jnp.maximum(xx[:, :128], xx[:, 128:]), 0.0)


def _s5(x):
    return _pc(
        _s5_body, grid=(B,),
        in_specs=[pl.BlockSpec((1, 64, 256), lambda b: (b, 0, 0))],
        out_specs=pl.BlockSpec((1, 64, 128), lambda b: (b, 0, 0)),
        out_shape=jax.ShapeDtypeStruct((B, 64, 128), jnp.float32),
    )(x)


def _s6_body(xt_ref, wp_ref, bp_ref, gf_ref, w1a_ref, w1b_ref, b1_ref,
             w2_ref, b2_ref, wo_ref, bo_ref, o_ref, acc_ref):
    j = pl.program_id(0)

    @pl.when(j == 0)
    def _():
        acc_ref[...] = jnp.zeros_like(acc_ref)

    acc_ref[...] += jnp.dot(xt_ref[...], wp_ref[...],
                            preferred_element_type=jnp.float32)

    @pl.when(j == pl.num_programs(0) - 1)
    def _():
        xt = jnp.maximum(acc_ref[...] + bp_ref[...], 0.0)
        y = jnp.dot(gf_ref[...], w1a_ref[...], preferred_element_type=jnp.float32)
        y = y + jnp.dot(xt, w1b_ref[...], preferred_element_type=jnp.float32)
        y = jnp.maximum(y + b1_ref[...], 0.0)
        y = jnp.maximum(jnp.dot(y, w2_ref[...], preferred_element_type=jnp.float32)
                        + b2_ref[...], 0.0)
        o_ref[...] = jnp.dot(y, wo_ref[...], preferred_element_type=jnp.float32) + bo_ref[...]


def _s6(xt_flat, wp, bp, gf, w1a, w1b, b1, w2, b2, wo, bo):
    KB = 2048
    return _pc(
        _s6_body, grid=(8192 // KB,),
        in_specs=[
            pl.BlockSpec((8, KB), lambda j: (0, j)),
            pl.BlockSpec((KB, 1024), lambda j: (j, 0)),
            pl.BlockSpec((1, 1024), lambda j: (0, 0)),
            pl.BlockSpec((8, 128), lambda j: (0, 0)),
            pl.BlockSpec((128, 1024), lambda j: (0, 0)),
            pl.BlockSpec((1024, 1024), lambda j: (0, 0)),
            pl.BlockSpec((1, 1024), lambda j: (0, 0)),
            pl.BlockSpec((1024, 256), lambda j: (0, 0)),
            pl.BlockSpec((1, 256), lambda j: (0, 0)),
            pl.BlockSpec((256, 128), lambda j: (0, 0)),
            pl.BlockSpec((1, 128), lambda j: (0, 0)),
        ],
        out_specs=pl.BlockSpec((8, 128), lambda j: (0, 0)),
        out_shape=jax.ShapeDtypeStruct((8, 128), jnp.float32),
        scratch_shapes=[pltpu.VMEM((8, 1024), jnp.float32)],
    )(xt_flat, wp, bp, gf, w1a, w1b, b1, w2, b2, wo, bo)


NSUB = 16
ES1 = EP // NSUB
ES2 = EP // (2 * NSUB)
NR = NP // NSUB
RNG = 1792
NQ = 3
RT = RNG // NSUB
BE = 592
NBLK = ES1 // BE
LCAP = BE + 64

_SC_PARAMS = pltpu.CompilerParams(
    needs_layout_passes=False, use_tc_tiling_on_sc=False)


def _rsqrt16(v):
    y = 1.0 / v
    for _ in range(18):
        y = y * (1.5 - 0.5 * v * y * y)
    return y


def _sca_body(asq_h, adq_h, src_h, dst_h, z0_h, alpha_h,
              sh_as, sh_ad, sh_z, sidx, didx, ga, gb, gz, gzs, pbuf, abuf):
    c = lax.axis_index("c")
    s = lax.axis_index("s")
    lanes = lax.iota(jnp.int32, 16)

    r0 = s * NR
    pltpu.sync_copy(asq_h.at[pl.ds(r0, NR)], sh_as.at[pl.ds(r0, NR)])
    pltpu.sync_copy(adq_h.at[pl.ds(r0, NR)], sh_ad.at[pl.ds(r0, NR)])
    pltpu.sync_copy(z0_h.at[pl.ds(r0, NR)], sh_z.at[pl.ds(r0, NR)])
    plsc.subcore_barrier()

    e0 = s * ES1
    pltpu.sync_copy(src_h.at[pl.ds(e0, ES1)], sidx)
    pltpu.sync_copy(dst_h.at[pl.ds(e0, ES1)], didx)

    def p1(j, carry):
        sv = sidx[pl.ds(j * 16, 16)]
        dv = didx[pl.ds(j * 16, 16)]
        pltpu.sync_copy(sh_as.at[sv], ga)
        pltpu.sync_copy(sh_ad.at[dv], gb)
        for r in range(16):
            av = ga[r] + gb[r]
            e = jnp.where(av > 0, av, 0.2 * av)
            pbuf[r] = jnp.exp(e)
        pltpu.sync_copy(pbuf, sh_z.at[dv], add=True)
        return carry

    lax.fori_loop(0, ES1 // 16, p1, 0)
    plsc.subcore_barrier()

    def pdis(t, carry):
        base = s * NR + t * 16
        pltpu.sync_copy(sh_z.at[pl.ds(base, 16)], gz)
        for r in range(16):
            v = gz[r]
            gz[r] = jnp.where(lanes == 15, _rsqrt16(v), v)
        pltpu.sync_copy(gz, sh_z.at[pl.ds(base, 16)])
        return carry

    lax.fori_loop(0, NR // 16, pdis, 0)
    plsc.subcore_barrier()

    e2 = c * (EP // 2) + s * ES2
    pltpu.sync_copy(src_h.at[pl.ds(e2, ES2)], sidx.at[pl.ds(0, ES2)])
    pltpu.sync_copy(dst_h.at[pl.ds(e2, ES2)], didx.at[pl.ds(0, ES2)])

    def p2(j, carry):
        sv = sidx[pl.ds(j * 16, 16)]
        dv = didx[pl.ds(j * 16, 16)]
        pltpu.sync_copy(sh_as.at[sv], ga)
        pltpu.sync_copy(sh_ad.at[dv], gb)
        pltpu.sync_copy(sh_z.at[dv], gz)
        pltpu.sync_copy(sh_z.at[sv], gzs)
        for r in range(16):
            av = ga[r] + gb[r]
            e = jnp.where(av > 0, av, 0.2 * av)
            p = jnp.exp(e)
            zv = gz[r]
            al = p / (zv + 1e-16)
            nv = gzs[r] * zv
            abuf[r] = jnp.where(lanes < 10, al, nv)
        pltpu.sync_copy(abuf, alpha_h.at[pl.ds(e2 + j * 16, 16)])
        return carry

    lax.fori_loop(0, ES2 // 16, p2, 0)


def _sca(asq, adq, srcp, dstp, z0):
    mesh = plsc.VectorSubcoreMesh(core_axis_name="c", subcore_axis_name="s")
    f = pl.kernel(
        _sca_body, mesh=mesh,
        out_type=jax.ShapeDtypeStruct((EP, 16), jnp.float32),
        scratch_types=[
            pltpu.VMEM_SHARED((NP, 16), jnp.float32),
            pltpu.VMEM_SHARED((NP, 16), jnp.float32),
            pltpu.VMEM_SHARED((NP, 16), jnp.float32),
            pltpu.VMEM((ES1,), jnp.int32),
            pltpu.VMEM((ES1,), jnp.int32),
            pltpu.VMEM((16, 16), jnp.float32),
            pltpu.VMEM((16, 16), jnp.float32),
            pltpu.VMEM((16, 16), jnp.float32),
            pltpu.VMEM((16, 16), jnp.float32),
            pltpu.VMEM((16, 16), jnp.float32),
            pltpu.VMEM((16, 16), jnp.float32),
        ],
        compiler_params=_SC_PARAMS)
    return f(asq, adq, srcp, dstp, z0)


def _scagg_body(table_h, alpha_h, src_h, dst_h, hm_h, z784_h, out_h,
                sh_acc, sblk, dblk, lsrc, loff, leid,
                hr0, hr1, ar0, ar1, hmv,
                sh0, sa0, sh1, sa1):
    c = lax.axis_index("c")
    s = lax.axis_index("s")
    lanes = lax.iota(jnp.int32, 16)
    pltpu.sync_copy(hm_h, hmv)
    e0 = s * ES1

    def init(t, carry):
        lsrc[pl.ds(t * 16, 16)] = lanes * 0
        loff[pl.ds(t * 16, 16)] = lanes * 0 + RNG
        leid[pl.ds(t * 16, 16)] = lanes * 0
        return carry

    lax.fori_loop(0, LCAP // 16, init, 0)

    def fire(g, hrb, arb, shb, sab):
        svv = lsrc[pl.ds(g * 16, 16)]
        evv = leid[pl.ds(g * 16, 16)]
        pltpu.async_copy(table_h.at[svv], hrb, shb)
        pltpu.async_copy(alpha_h.at[evv], arb, sab)

    def waitg(hrb, arb, shb, sab):
        pltpu.make_async_copy(table_h.at[pl.ds(0, 16)], hrb, shb).wait()
        pltpu.make_async_copy(alpha_h.at[pl.ds(0, 16)], arb, sab).wait()

    def compute(g, hrb, arb):
        ovv = loff[pl.ds(g * 16, 16)]

        def ci(i, carry):
            rvec = lanes * 0 + i
            for v in range(F // 16):
                hv = hrb[i, pl.ds(v * 16, 16)]
                am = plsc.load_gather(arb, [rvec, hmv[v]])
                hrb[i, pl.ds(v * 16, 16)] = hv * am
            return carry

        lax.fori_loop(0, 16, ci, 0)
        pltpu.sync_copy(hrb, sh_acc.at[ovv], add=True)

    for q in range(NQ):
        lo = (NQ * c + q) * RNG

        def zz(t, carry):
            pltpu.sync_copy(z784_h, sh_acc.at[pl.ds(s * RT + t * 16, 16)])
            return carry

        lax.fori_loop(0, RT // 16, zz, 0)
        plsc.subcore_barrier()

        def blk_body(b, carry):
            eb = e0 + b * BE
            pltpu.sync_copy(src_h.at[pl.ds(eb, BE)], sblk)
            pltpu.sync_copy(dst_h.at[pl.ds(eb, BE)], dblk)

            def scan(j, cnt):
                dv = dblk[pl.ds(j * 16, 16)]
                sv = sblk[pl.ds(j * 16, 16)]
                m = (dv >= lo) & (dv < lo + RNG)
                mi = m.astype(jnp.int32)
                pref = plsc.cumsum(mi)
                pos = cnt + pref - mi
                plsc.store_scatter(lsrc, [pos], sv, mask=m)
                plsc.store_scatter(loff, [pos], dv - lo, mask=m)
                plsc.store_scatter(leid, [pos], eb + j * 16 + lanes, mask=m)
                return cnt + jnp.sum(mi)

            cnt = lax.fori_loop(0, BE // 16, scan, 0)
            allm = lanes < 16
            for pad in range(2):
                ppos = cnt + pad * 16 + lanes
                plsc.store_scatter(lsrc, [ppos], lanes * 0 + N, mask=allm)
                plsc.store_scatter(loff, [ppos], lanes * 0 + RNG, mask=allm)
                plsc.store_scatter(leid, [ppos], lanes * 0, mask=allm)

            ng2 = (cnt + 31) // 32
            fire(0, hr0, ar0, sh0, sa0)

            def lp(g2, c2):
                g = g2 * 2
                waitg(hr0, ar0, sh0, sa0)
                fire(g + 1, hr1, ar1, sh1, sa1)
                compute(g, hr0, ar0)
                waitg(hr1, ar1, sh1, sa1)
                fire(g + 2, hr0, ar0, sh0, sa0)
                compute(g + 1, hr1, ar1)
                return c2

            lax.fori_loop(0, ng2, lp, 0)
            waitg(hr0, ar0, sh0, sa0)
            return carry

        lax.fori_loop(0, NBLK, blk_body, 0)
        plsc.subcore_barrier()

        rows_lo = lo + s * RT
        nch = jnp.clip((NP - rows_lo) // 16, 0, RT // 16)

        def wb(t, carry):
            r = rows_lo + t * 16
            pltpu.sync_copy(sh_acc.at[pl.ds(s * RT + t * 16, 16)],
                            out_h.at[pl.ds(r, 16)])
            return carry

        lax.fori_loop(0, nch, wb, 0)
        plsc.subcore_barrier()


def _scagg(table, alpha, srcp, dstp, hm, z784):
    mesh = plsc.VectorSubcoreMesh(core_axis_name="c", subcore_axis_name="s")
    f = pl.kernel(
        _scagg_body, mesh=mesh,
        out_type=jax.ShapeDtypeStruct((NP, F), jnp.float32),
        scratch_types=[
            pltpu.VMEM_SHARED((RNG + 8, F), jnp.float32),
            pltpu.VMEM((BE,), jnp.int32),
            pltpu.VMEM((BE,), jnp.int32),
            pltpu.VMEM((LCAP,), jnp.int32),
            pltpu.VMEM((LCAP,), jnp.int32),
            pltpu.VMEM((LCAP,), jnp.int32),
            pltpu.VMEM((16, F), jnp.float32),
            pltpu.VMEM((16, F), jnp.float32),
            pltpu.VMEM((16, 16), jnp.float32),
            pltpu.VMEM((16, 16), jnp.float32),
            pltpu.VMEM((49, 16), jnp.int32),
            pltpu.SemaphoreType.DMA,
            pltpu.SemaphoreType.DMA,
            pltpu.SemaphoreType.DMA,
            pltpu.SemaphoreType.DMA,
        ],
        compiler_params=_SC_PARAMS)
    return f(table, alpha, srcp, dstp, hm, z784)


def kernel(x, seq_embed, W_gat, att_src, att_dst, b_gat, W_gcn, b_gcn,
           W_fcg1, b_fcg1, W_fcg2, b_fcg2, Wc1, bc1, Wc2, bc2, Wc3, bc3,
           Wc4, bc4, W_fcxt, b_fcxt, W_fc1, b_fc1, W_fc2, b_fc2,
           W_out, b_out, edge_index, batch):
    f32 = jnp.float32

    x_p = jnp.zeros((NP, 128), f32).at[:N, :D].set(x)
    w_gat_p = jnp.zeros((128, F), f32).at[:D, :H * D].set(W_gat)
    rows = jnp.arange(H * D)
    a_s = jnp.zeros((F, 16), f32).at[rows, rows // D].set(att_src.reshape(-1))
    a_d = jnp.zeros((F, 16), f32).at[rows, rows // D].set(att_dst.reshape(-1))
    b_gat_p = jnp.zeros((1, F), f32).at[0, :H * D].set(b_gat)
    w_gcn_p = jnp.zeros((F, F), f32).at[:H * D, :H * D].set(W_gcn)
    b_gcn_p = jnp.zeros((1, F), f32).at[0, :H * D].set(b_gcn)

    i32 = jnp.int32
    srcp = jnp.full((EP,), N, i32).at[:E + N].set(
        jnp.concatenate([edge_index[0], jnp.arange(N, dtype=i32)]))
    dstp = jnp.full((EP,), N, i32).at[:E + N].set(
        jnp.concatenate([edge_index[1], jnp.arange(N, dtype=i32)]))

    h, as_q, ad_q = _k1(x_p, w_gat_p, a_s, a_d)

    z0 = jnp.zeros((NP, 16), f32)
    z784 = jnp.zeros((16, F), f32)
    hm_gat = jnp.minimum(jnp.arange(F, dtype=i32) // D, 9).reshape(49, 16)
    hm_gcn = jnp.full((49, 16), 15, i32)
    alpha = _sca(as_q, ad_q, srcp, dstp, z0)
    gat_out = _scagg(h, alpha, srcp, dstp, hm_gat, z784)

    h2 = _k2(gat_out, b_gat_p, w_gcn_p)

    gcn_out = _scagg(h2, alpha, srcp, dstp, hm_gcn, z784)

    onehot = jnp.zeros((NP, 8), f32).at[jnp.arange(N), batch].set(1.0)
    gm, gs = _k3(gcn_out, b_gcn_p, onehot)
    cnt = jnp.maximum(jnp.sum(onehot, axis=0), 1.0)
    ga = gs / cnt[:, None]
    G1 = 1536
    w1a = jnp.zeros((F, G1), f32).at[:H * D, :1500].set(W_fcg1[:H * D])
    w1b = jnp.zeros((F, G1), f32).at[:H * D, :1500].set(W_fcg1[H * D:])
    b1 = jnp.zeros((1, G1), f32).at[0, :1500].set(b_fcg1)
    w2 = jnp.zeros((G1, 128), f32).at[:1500].set(W_fcg2)
    gfeat = _k4(gm, ga, w1a, w1b, b1, w2, b_fcg2.reshape(1, 128))

    xs_p = jnp.zeros((B, 1032, SD), f32).at[:, :L, :].set(seq_embed)
    y1 = _conv_stage(xs_p, jnp.transpose(Wc1, (2, 1, 0)), bc1, 5, 4, 256, SD, 256, True)
    x2 = jnp.zeros((B, 520, 512), f32).at[:, :512, :].set(y1.reshape(B, 512, 512))
    y2 = _conv_stage(x2, jnp.transpose(Wc2, (2, 1, 0)), bc2, 5, 2, 256, 256, SD, False)
    x3 = jnp.zeros((B, 264, 2048), f32).at[:, :256, :].set(y2.reshape(B, 256, 2048))
    y3 = _conv_stage(x3, jnp.transpose(Wc3, (2, 1, 0)), bc3, 5, 1, 256, SD, 256, False)
    x4 = jnp.zeros((B, 136, 512), f32).at[:, :128, :].set(y3.reshape(B, 128, 512))
    y4 = _conv_stage(x4, jnp.transpose(Wc4, (2, 1, 0)), bc4, 3, 1, 128, 256, 128, False)
    pooled4 = _s5(y4.reshape(B, 64, 256))
    xt_flat = jnp.zeros((8, 8192), f32).at[:B].set(pooled4.reshape(B, 8192))

    lidx = jnp.arange(8192) // 128
    cidx = jnp.arange(8192) % 128
    srcrow = jnp.where(lidx < 61, cidx * 61 + lidx, 0)
    wp = jnp.where((lidx < 61)[:, None], W_fcxt[srcrow], 0.0)
    gf = jnp.zeros((8, 128), f32).at[:B].set(gfeat[:B])
    w1a_f = W_fc1[:128]
    w1b_f = W_fc1[128:]
    wo = jnp.zeros((256, 128), f32).at[:, 0].set(W_out[:, 0])
    bo = jnp.zeros((1, 128), f32).at[0, 0].set(b_out[0])
    out = _s6(xt_flat, wp, b_fcxt.reshape(1, 1024), gf, w1a_f, w1b_f,
              b_fc1.reshape(1, 1024), W_fc2, b_fc2.reshape(1, 256),
              wo, bo)
    return out[:B, :1]


# speedup vs baseline: 4.2841x; 1.2834x over previous
import functools

import jax
import jax.numpy as jnp
from jax import lax
from jax.experimental import pallas as pl
from jax.experimental.pallas import tpu as pltpu
from jax.experimental.pallas import tpu_sc as plsc

N = 10000
E = 160000
D = 78
H = 10
B = 4
L = 1020
SD = 1024

NP = 10240
BN = 256
F = 784
EP = 170496

_INTERPRET = False


def _pc(body, grid, in_specs, out_specs, out_shape, **kw):
    return pl.pallas_call(
        body, grid=grid, in_specs=in_specs, out_specs=out_specs,
        out_shape=out_shape, interpret=_INTERPRET, **kw)


def _k1_body(x_ref, w_ref, as_ref, ad_ref, h_ref, asq_ref, adq_ref):
    h = jnp.dot(x_ref[...], w_ref[...], preferred_element_type=jnp.float32)
    h_ref[...] = h
    asq_ref[...] = jnp.dot(h, as_ref[...], preferred_element_type=jnp.float32)
    adq_ref[...] = jnp.dot(h, ad_ref[...], preferred_element_type=jnp.float32)


def _k1(x_p, w_p, a_s, a_d):
    return _pc(
        _k1_body, grid=(NP // BN,),
        in_specs=[
            pl.BlockSpec((BN, 128), lambda i: (i, 0)),
            pl.BlockSpec((128, F), lambda i: (0, 0)),
            pl.BlockSpec((F, 16), lambda i: (0, 0)),
            pl.BlockSpec((F, 16), lambda i: (0, 0)),
        ],
        out_specs=[
            pl.BlockSpec((BN, F), lambda i: (i, 0)),
            pl.BlockSpec((BN, 16), lambda i: (i, 0)),
            pl.BlockSpec((BN, 16), lambda i: (i, 0)),
        ],
        out_shape=[
            jax.ShapeDtypeStruct((NP, F), jnp.float32),
            jax.ShapeDtypeStruct((NP, 16), jnp.float32),
            jax.ShapeDtypeStruct((NP, 16), jnp.float32),
        ],
    )(x_p, w_p, a_s, a_d)


def _k2_body(g_ref, b_ref, w_ref, z_ref, o_ref):
    g = jnp.maximum(g_ref[...] + b_ref[...], 0.0)
    dis = z_ref[:, 15:16]
    o_ref[...] = jnp.dot(g, w_ref[...], preferred_element_type=jnp.float32) * dis


def _k2(gat_out, b_gat_p, w_gcn_p, zq):
    return _pc(
        _k2_body, grid=(NP // BN,),
        in_specs=[
            pl.BlockSpec((BN, F), lambda i: (i, 0)),
            pl.BlockSpec((1, F), lambda i: (0, 0)),
            pl.BlockSpec((F, F), lambda i: (0, 0)),
            pl.BlockSpec((BN, 16), lambda i: (i, 0)),
        ],
        out_specs=pl.BlockSpec((BN, F), lambda i: (i, 0)),
        out_shape=jax.ShapeDtypeStruct((NP, F), jnp.float32),
    )(gat_out, b_gat_p, w_gcn_p, zq)


def _k3_body(g_ref, b_ref, oh_ref, z_ref, gm_ref, gs_ref):
    i = pl.program_id(0)

    @pl.when(i == 0)
    def _():
        gm_ref[...] = jnp.full_like(gm_ref, -1e30)
        gs_ref[...] = jnp.zeros_like(gs_ref)

    g = jnp.maximum(g_ref[...] * z_ref[:, 15:16] + b_ref[...], 0.0)
    oh = oh_ref[...]
    for q in range(B):
        col = oh[:, q:q + 1]
        m = g * col + (col - 1.0) * 1e30
        gm_ref[q:q + 1, :] = jnp.maximum(
            gm_ref[q:q + 1, :], jnp.max(m, axis=0, keepdims=True))
        gs_ref[q:q + 1, :] = gs_ref[q:q + 1, :] + jnp.sum(
            g * col, axis=0, keepdims=True)


def _k3(gcn_out, b_gcn_p, onehot, zq):
    return _pc(
        _k3_body, grid=(NP // BN,),
        in_specs=[
            pl.BlockSpec((BN, F), lambda i: (i, 0)),
            pl.BlockSpec((1, F), lambda i: (0, 0)),
            pl.BlockSpec((BN, 8), lambda i: (i, 0)),
            pl.BlockSpec((BN, 16), lambda i: (i, 0)),
        ],
        out_specs=[
            pl.BlockSpec((8, F), lambda i: (0, 0)),
            pl.BlockSpec((8, F), lambda i: (0, 0)),
        ],
        out_shape=[
            jax.ShapeDtypeStruct((8, F), jnp.float32),
            jax.ShapeDtypeStruct((8, F), jnp.float32),
        ],
    )(gcn_out, b_gcn_p, onehot, zq)


def _k4_body(gm_ref, ga_ref, w1a_ref, w1b_ref, b1_ref, w2_ref, b2_ref, o_ref):
    y = jnp.dot(gm_ref[...], w1a_ref[...], preferred_element_type=jnp.float32)
    y = y + jnp.dot(ga_ref[...], w1b_ref[...], preferred_element_type=jnp.float32)
    y = jnp.maximum(y + b1_ref[...], 0.0)
    o_ref[...] = jnp.dot(y, w2_ref[...], preferred_element_type=jnp.float32) + b2_ref[...]


def _k4(gm, ga, w1a, w1b, b1, w2, b2):
    G1 = 1536
    return _pc(
        _k4_body, grid=(1,),
        in_specs=[
            pl.BlockSpec((8, F), lambda i: (0, 0)),
            pl.BlockSpec((8, F), lambda i: (0, 0)),
            pl.BlockSpec((F, G1), lambda i: (0, 0)),
            pl.BlockSpec((F, G1), lambda i: (0, 0)),
            pl.BlockSpec((1, G1), lambda i: (0, 0)),
            pl.BlockSpec((G1, 128), lambda i: (0, 0)),
            pl.BlockSpec((1, 128), lambda i: (0, 0)),
        ],
        out_specs=pl.BlockSpec((8, 128), lambda i: (0, 0)),
        out_shape=jax.ShapeDtypeStruct((8, 128), jnp.float32),
    )(gm, ga, w1a, w1b, b1, w2, b2)


def _conv_body(x_ref, w_ref, b_ref, o_ref, *, k, lb, cin, first):
    i = pl.program_id(1)
    xs_big = x_ref[0, pl.ds(i * lb, lb + 8), :]
    if not first:
        xs_big = jnp.maximum(jnp.maximum(xs_big[:, :cin], xs_big[:, cin:]), 0.0)
    acc = jnp.broadcast_to(b_ref[...], o_ref.shape[1:]).astype(jnp.float32)
    for kk in range(k):
        acc = acc + jnp.dot(xs_big[kk:kk + lb], w_ref[kk],
                            preferred_element_type=jnp.float32)
    o_ref[0] = acc


def _conv_stage(x, w, bias, k, nb, lb, cin, cout, first):
    lout = nb * lb
    return _pc(
        functools.partial(_conv_body, k=k, lb=lb, cin=cin, first=first),
        grid=(B, nb),
        in_specs=[
            pl.BlockSpec((1,) + x.shape[1:], lambda b, i: (b, 0, 0)),
            pl.BlockSpec(w.shape, lambda b, i: (0, 0, 0)),
            pl.BlockSpec((1, cout), lambda b, i: (0, 0)),
        ],
        out_specs=pl.BlockSpec((1, lb, cout), lambda b, i: (b, i, 0)),
        out_shape=jax.ShapeDtypeStruct((B, lout, cout), jnp.float32),
    )(x, w, bias.reshape(1, cout))


def _s5_body(x_ref, o_ref):
    xx = x_ref[0]
    o_ref[0] = jnp.maximum(jnp.maximum(xx[:, :128], xx[:, 128:]), 0.0)


def _s5(x):
    return _pc(
        _s5_body, grid=(B,),
        in_specs=[pl.BlockSpec((1, 64, 256), lambda b: (b, 0, 0))],
        out_specs=pl.BlockSpec((1, 64, 128), lambda b: (b, 0, 0)),
        out_shape=jax.ShapeDtypeStruct((B, 64, 128), jnp.float32),
    )(x)


def _s6_body(xt_ref, wp_ref, bp_ref, gf_ref, w1a_ref, w1b_ref, b1_ref,
             w2_ref, b2_ref, wo_ref, bo_ref, o_ref, acc_ref):
    j = pl.program_id(0)

    @pl.when(j == 0)
    def _():
        acc_ref[...] = jnp.zeros_like(acc_ref)

    acc_ref[...] += jnp.dot(xt_ref[...], wp_ref[...],
                            preferred_element_type=jnp.float32)

    @pl.when(j == pl.num_programs(0) - 1)
    def _():
        xt = jnp.maximum(acc_ref[...] + bp_ref[...], 0.0)
        y = jnp.dot(gf_ref[...], w1a_ref[...], preferred_element_type=jnp.float32)
        y = y + jnp.dot(xt, w1b_ref[...], preferred_element_type=jnp.float32)
        y = jnp.maximum(y + b1_ref[...], 0.0)
        y = jnp.maximum(jnp.dot(y, w2_ref[...], preferred_element_type=jnp.float32)
                        + b2_ref[...], 0.0)
        o_ref[...] = jnp.dot(y, wo_ref[...], preferred_element_type=jnp.float32) + bo_ref[...]


def _s6(xt_flat, wp, bp, gf, w1a, w1b, b1, w2, b2, wo, bo):
    KB = 2048
    return _pc(
        _s6_body, grid=(8192 // KB,),
        in_specs=[
            pl.BlockSpec((8, KB), lambda j: (0, j)),
            pl.BlockSpec((KB, 1024), lambda j: (j, 0)),
            pl.BlockSpec((1, 1024), lambda j: (0, 0)),
            pl.BlockSpec((8, 128), lambda j: (0, 0)),
            pl.BlockSpec((128, 1024), lambda j: (0, 0)),
            pl.BlockSpec((1024, 1024), lambda j: (0, 0)),
            pl.BlockSpec((1, 1024), lambda j: (0, 0)),
            pl.BlockSpec((1024, 256), lambda j: (0, 0)),
            pl.BlockSpec((1, 256), lambda j: (0, 0)),
            pl.BlockSpec((256, 128), lambda j: (0, 0)),
            pl.BlockSpec((1, 128), lambda j: (0, 0)),
        ],
        out_specs=pl.BlockSpec((8, 128), lambda j: (0, 0)),
        out_shape=jax.ShapeDtypeStruct((8, 128), jnp.float32),
        scratch_shapes=[pltpu.VMEM((8, 1024), jnp.float32)],
    )(xt_flat, wp, bp, gf, w1a, w1b, b1, w2, b2, wo, bo)


NSUB = 16
ES1 = EP // NSUB
ES2 = EP // (2 * NSUB)
NR = NP // NSUB
RNG = 1792
NQ = 3
RT = RNG // NSUB
BE = 592
NBLK = ES1 // BE
LCAP = BE + 64

_SC_PARAMS = pltpu.CompilerParams(
    needs_layout_passes=False, use_tc_tiling_on_sc=False)


def _rsqrt16(v):
    y = 1.0 / v
    for _ in range(18):
        y = y * (1.5 - 0.5 * v * y * y)
    return y


def _sca_body(asq_h, adq_h, src_h, dst_h, z0_h, alpha_h, zout_h,
              sh_as, sh_ad, sh_z, sidx, didx,
              ga0, gb0, gz0, gzs0, pbuf0, ga1, gb1, gz1, gzs1, pbuf1,
              sg0, sg1, sw0, sw1):
    c = lax.axis_index("c")
    s = lax.axis_index("s")
    lanes = lax.iota(jnp.int32, 16)

    r0 = s * NR
    pltpu.sync_copy(asq_h.at[pl.ds(r0, NR)], sh_as.at[pl.ds(r0, NR)])
    pltpu.sync_copy(adq_h.at[pl.ds(r0, NR)], sh_ad.at[pl.ds(r0, NR)])
    pltpu.sync_copy(z0_h.at[pl.ds(r0, NR)], sh_z.at[pl.ds(r0, NR)])
    plsc.subcore_barrier()

    e0 = s * ES1
    pltpu.sync_copy(src_h.at[pl.ds(e0, ES1)], sidx.at[pl.ds(0, ES1)])
    pltpu.sync_copy(dst_h.at[pl.ds(e0, ES1)], didx.at[pl.ds(0, ES1)])
    for t in range(2):
        sidx[pl.ds(ES1 + t * 16, 16)] = lanes * 0
        didx[pl.ds(ES1 + t * 16, 16)] = lanes * 0

    def fire1(j, gax, gbx, sgx):
        sv = sidx[pl.ds(j * 16, 16)]
        dv = didx[pl.ds(j * 16, 16)]
        pltpu.async_copy(sh_as.at[sv], gax, sgx)
        pltpu.async_copy(sh_ad.at[dv], gbx, sgx)

    def wait1(gax, gbx, sgx):
        pltpu.make_async_copy(asq_h.at[pl.ds(0, 16)], gax, sgx).wait()
        pltpu.make_async_copy(adq_h.at[pl.ds(0, 16)], gbx, sgx).wait()

    def comp1(j, gax, gbx, pbx):
        dv = didx[pl.ds(j * 16, 16)]
        for r in range(16):
            av = gax[r] + gbx[r]
            e = jnp.where(av > 0, av, 0.2 * av)
            pbx[r] = jnp.exp(e)
        pltpu.sync_copy(pbx, sh_z.at[dv], add=True)

    fire1(0, ga0, gb0, sg0)

    def p1(t, carry):
        j = t * 2
        wait1(ga0, gb0, sg0)
        fire1(j + 1, ga1, gb1, sg1)
        comp1(j, ga0, gb0, pbuf0)
        wait1(ga1, gb1, sg1)
        fire1(j + 2, ga0, gb0, sg0)
        comp1(j + 1, ga1, gb1, pbuf1)
        return carry

    lax.fori_loop(0, ES1 // 32, p1, 0)
    wait1(ga0, gb0, sg0)
    plsc.subcore_barrier()

    def pdis(t, carry):
        base = s * NR + t * 16
        pltpu.sync_copy(sh_z.at[pl.ds(base, 16)], gz0)
        for r in range(16):
            v = gz0[r]
            dis = jnp.where(v > 0, _rsqrt16(v), 0.0)
            gz0[r] = jnp.where(lanes == 15, dis, v)
        pltpu.sync_copy(gz0, sh_z.at[pl.ds(base, 16)])
        return carry

    lax.fori_loop(0, NR // 16, pdis, 0)
    plsc.subcore_barrier()

    @pl.when(c == 0)
    def _():
        pltpu.sync_copy(sh_z.at[pl.ds(r0, NR)], zout_h.at[pl.ds(r0, NR)])

    e2 = c * (EP // 2) + s * ES2
    pltpu.sync_copy(src_h.at[pl.ds(e2, ES2)], sidx.at[pl.ds(0, ES2)])
    pltpu.sync_copy(dst_h.at[pl.ds(e2, ES2)], didx.at[pl.ds(0, ES2)])
    for t in range(2):
        sidx[pl.ds(ES2 + t * 16, 16)] = lanes * 0
        didx[pl.ds(ES2 + t * 16, 16)] = lanes * 0

    def fire2(j, gax, gbx, gzx, gzsx, sgx):
        sv = sidx[pl.ds(j * 16, 16)]
        dv = didx[pl.ds(j * 16, 16)]
        pltpu.async_copy(sh_as.at[sv], gax, sgx)
        pltpu.async_copy(sh_ad.at[dv], gbx, sgx)
        pltpu.async_copy(sh_z.at[dv], gzx, sgx)

    def wait2(gax, gbx, gzx, gzsx, sgx):
        pltpu.make_async_copy(asq_h.at[pl.ds(0, 16)], gax, sgx).wait()
        pltpu.make_async_copy(adq_h.at[pl.ds(0, 16)], gbx, sgx).wait()
        pltpu.make_async_copy(z0_h.at[pl.ds(0, 16)], gzx, sgx).wait()

    def comp2(j, gax, gbx, gzx, gzsx, pbx, swx):
        for r in range(16):
            av = gax[r] + gbx[r]
            e = jnp.where(av > 0, av, 0.2 * av)
            p = jnp.exp(e)
            al = p / (gzx[r] + 1e-16)
            pbx[r] = al
        pltpu.async_copy(pbx, alpha_h.at[pl.ds(e2 + j * 16, 16)], swx)

    fire2(0, ga0, gb0, gz0, gzs0, sg0)

    def p2(t, carry):
        j = t * 2
        wait2(ga0, gb0, gz0, gzs0, sg0)
        fire2(j + 1, ga1, gb1, gz1, gzs1, sg1)

        @pl.when(t > 0)
        def _():
            pltpu.make_async_copy(z0_h.at[pl.ds(0, 16)], pbuf0, sw0).wait()

        comp2(j, ga0, gb0, gz0, gzs0, pbuf0, sw0)
        wait2(ga1, gb1, gz1, gzs1, sg1)
        fire2(j + 2, ga0, gb0, gz0, gzs0, sg0)

        @pl.when(t > 0)
        def _():
            pltpu.make_async_copy(z0_h.at[pl.ds(0, 16)], pbuf1, sw1).wait()

        comp2(j + 1, ga1, gb1, gz1, gzs1, pbuf1, sw1)
        return carry

    lax.fori_loop(0, ES2 // 32, p2, 0)
    wait2(ga0, gb0, gz0, gzs0, sg0)
    pltpu.make_async_copy(z0_h.at[pl.ds(0, 16)], pbuf0, sw0).wait()
    pltpu.make_async_copy(z0_h.at[pl.ds(0, 16)], pbuf1, sw1).wait()
    comp2(ES2 // 16 - 1, ga0, gb0, gz0, gzs0, pbuf0, sw0)
    pltpu.make_async_copy(z0_h.at[pl.ds(0, 16)], pbuf0, sw0).wait()


def _sca(asq, adq, srcp, dstp, z0):
    mesh = plsc.VectorSubcoreMesh(core_axis_name="c", subcore_axis_name="s")
    f = pl.kernel(
        _sca_body, mesh=mesh,
        out_type=[jax.ShapeDtypeStruct((EP, 16), jnp.float32),
                  jax.ShapeDtypeStruct((NP, 16), jnp.float32)],
        scratch_types=[
            pltpu.VMEM_SHARED((NP, 16), jnp.float32),
            pltpu.VMEM_SHARED((NP, 16), jnp.float32),
            pltpu.VMEM_SHARED((NP, 16), jnp.float32),
            pltpu.VMEM((ES1 + 32,), jnp.int32),
            pltpu.VMEM((ES1 + 32,), jnp.int32),
            pltpu.VMEM((16, 16), jnp.float32),
            pltpu.VMEM((16, 16), jnp.float32),
            pltpu.VMEM((16, 16), jnp.float32),
            pltpu.VMEM((16, 16), jnp.float32),
            pltpu.VMEM((16, 16), jnp.float32),
            pltpu.VMEM((16, 16), jnp.float32),
            pltpu.VMEM((16, 16), jnp.float32),
            pltpu.VMEM((16, 16), jnp.float32),
            pltpu.VMEM((16, 16), jnp.float32),
            pltpu.VMEM((16, 16), jnp.float32),
            pltpu.SemaphoreType.DMA,
            pltpu.SemaphoreType.DMA,
            pltpu.SemaphoreType.DMA,
            pltpu.SemaphoreType.DMA,
        ],
        compiler_params=_SC_PARAMS)
    return f(asq, adq, srcp, dstp, z0)


def _scagg_body(table_h, alpha_h, src_h, dst_h, hm_h, z784_h, out_h,
                sh_acc, sblk, dblk, lsrc, loff, leid,
                hr0, hr1, ar0, ar1, hmv,
                sh0, sa0, sh1, sa1):
    c = lax.axis_index("c")
    s = lax.axis_index("s")
    lanes = lax.iota(jnp.int32, 16)
    pltpu.sync_copy(hm_h, hmv)
    e0 = s * ES1

    def init(t, carry):
        lsrc[pl.ds(t * 16, 16)] = lanes * 0
        loff[pl.ds(t * 16, 16)] = lanes * 0 + RNG
        leid[pl.ds(t * 16, 16)] = lanes * 0
        return carry

    lax.fori_loop(0, LCAP // 16, init, 0)

    def fire(g, hrb, arb, shb, sab):
        svv = lsrc[pl.ds(g * 16, 16)]
        evv = leid[pl.ds(g * 16, 16)]
        pltpu.async_copy(table_h.at[svv], hrb, shb)
        pltpu.async_copy(alpha_h.at[evv], arb, sab)

    def waitg(hrb, arb, shb, sab):
        pltpu.make_async_copy(table_h.at[pl.ds(0, 16)], hrb, shb).wait()
        pltpu.make_async_copy(alpha_h.at[pl.ds(0, 16)], arb, sab).wait()

    def compute(g, hrb, arb):
        ovv = loff[pl.ds(g * 16, 16)]

        def ci(i, carry):
            rvec = lanes * 0 + i
            for v in range(F // 16):
                hv = hrb[i, pl.ds(v * 16, 16)]
                am = plsc.load_gather(arb, [rvec, hmv[v]])
                hrb[i, pl.ds(v * 16, 16)] = hv * am
            return carry

        lax.fori_loop(0, 16, ci, 0)
        pltpu.sync_copy(hrb, sh_acc.at[ovv], add=True)

    for q in range(NQ):
        lo = (NQ * c + q) * RNG

        def zz(t, carry):
            pltpu.sync_copy(z784_h, sh_acc.at[pl.ds(s * RT + t * 16, 16)])
            return carry

        lax.fori_loop(0, RT // 16, zz, 0)
        plsc.subcore_barrier()

        def blk_body(b, carry):
            eb = e0 + b * BE
            pltpu.sync_copy(src_h.at[pl.ds(eb, BE)], sblk)
            pltpu.sync_copy(dst_h.at[pl.ds(eb, BE)], dblk)

            def scan(j, cnt):
                dv = dblk[pl.ds(j * 16, 16)]
                sv = sblk[pl.ds(j * 16, 16)]
                m = (dv >= lo) & (dv < lo + RNG)
                mi = m.astype(jnp.int32)
                pref = plsc.cumsum(mi)
                pos = cnt + pref - mi
                plsc.store_scatter(lsrc, [pos], sv, mask=m)
                plsc.store_scatter(loff, [pos], dv - lo, mask=m)
                plsc.store_scatter(leid, [pos], eb + j * 16 + lanes, mask=m)
                return cnt + jnp.sum(mi)

            cnt = lax.fori_loop(0, BE // 16, scan, 0)
            allm = lanes < 16
            for pad in range(2):
                ppos = cnt + pad * 16 + lanes
                plsc.store_scatter(lsrc, [ppos], lanes * 0 + N, mask=allm)
                plsc.store_scatter(loff, [ppos], lanes * 0 + RNG, mask=allm)
                plsc.store_scatter(leid, [ppos], lanes * 0, mask=allm)

            ng2 = (cnt + 31) // 32
            fire(0, hr0, ar0, sh0, sa0)

            def lp(g2, c2):
                g = g2 * 2
                waitg(hr0, ar0, sh0, sa0)
                fire(g + 1, hr1, ar1, sh1, sa1)
                compute(g, hr0, ar0)
                waitg(hr1, ar1, sh1, sa1)
                fire(g + 2, hr0, ar0, sh0, sa0)
                compute(g + 1, hr1, ar1)
                return c2

            lax.fori_loop(0, ng2, lp, 0)
            waitg(hr0, ar0, sh0, sa0)
            return carry

        lax.fori_loop(0, NBLK, blk_body, 0)
        plsc.subcore_barrier()

        rows_lo = lo + s * RT
        nch = jnp.clip((NP - rows_lo) // 16, 0, RT // 16)

        def wb(t, carry):
            r = rows_lo + t * 16
            pltpu.sync_copy(sh_acc.at[pl.ds(s * RT + t * 16, 16)],
                            out_h.at[pl.ds(r, 16)])
            return carry

        lax.fori_loop(0, nch, wb, 0)
        plsc.subcore_barrier()


def _scagg(table, alpha, srcp, dstp, hm, z784):
    mesh = plsc.VectorSubcoreMesh(core_axis_name="c", subcore_axis_name="s")
    f = pl.kernel(
        _scagg_body, mesh=mesh,
        out_type=jax.ShapeDtypeStruct((NP, F), jnp.float32),
        scratch_types=[
            pltpu.VMEM_SHARED((RNG + 8, F), jnp.float32),
            pltpu.VMEM((BE,), jnp.int32),
            pltpu.VMEM((BE,), jnp.int32),
            pltpu.VMEM((LCAP,), jnp.int32),
            pltpu.VMEM((LCAP,), jnp.int32),
            pltpu.VMEM((LCAP,), jnp.int32),
            pltpu.VMEM((16, F), jnp.float32),
            pltpu.VMEM((16, F), jnp.float32),
            pltpu.VMEM((16, 16), jnp.float32),
            pltpu.VMEM((16, 16), jnp.float32),
            pltpu.VMEM((49, 16), jnp.int32),
            pltpu.SemaphoreType.DMA,
            pltpu.SemaphoreType.DMA,
            pltpu.SemaphoreType.DMA,
            pltpu.SemaphoreType.DMA,
        ],
        compiler_params=_SC_PARAMS)
    return f(table, alpha, srcp, dstp, hm, z784)


def _scdma_body(table_h, src_h, dst_h, z784_h, out_h,
                sh_acc, sblk, dblk, lsrc, loff, hr0, hr1, sh0, sh1):
    c = lax.axis_index("c")
    s = lax.axis_index("s")
    lanes = lax.iota(jnp.int32, 16)
    e0 = s * ES1

    def init(t, carry):
        lsrc[pl.ds(t * 16, 16)] = lanes * 0
        loff[pl.ds(t * 16, 16)] = lanes * 0 + RNG
        return carry

    lax.fori_loop(0, LCAP // 16, init, 0)

    def fire(g, hrb, shb):
        svv = lsrc[pl.ds(g * 16, 16)]
        pltpu.async_copy(table_h.at[svv], hrb, shb)

    def waitg(hrb, shb):
        pltpu.make_async_copy(table_h.at[pl.ds(0, 16)], hrb, shb).wait()

    def scat(g, hrb):
        ovv = loff[pl.ds(g * 16, 16)]
        pltpu.sync_copy(hrb, sh_acc.at[ovv], add=True)

    for q in range(NQ):
        lo = (NQ * c + q) * RNG

        def zz(t, carry):
            pltpu.sync_copy(z784_h, sh_acc.at[pl.ds(s * RT + t * 16, 16)])
            return carry

        lax.fori_loop(0, RT // 16, zz, 0)
        plsc.subcore_barrier()

        def blk_body(b, carry):
            eb = e0 + b * BE
            pltpu.sync_copy(src_h.at[pl.ds(eb, BE)], sblk)
            pltpu.sync_copy(dst_h.at[pl.ds(eb, BE)], dblk)

            def scan(j, cnt):
                dv = dblk[pl.ds(j * 16, 16)]
                sv = sblk[pl.ds(j * 16, 16)]
                m = (dv >= lo) & (dv < lo + RNG)
                mi = m.astype(jnp.int32)
                pref = plsc.cumsum(mi)
                pos = cnt + pref - mi
                plsc.store_scatter(lsrc, [pos], sv, mask=m)
                plsc.store_scatter(loff, [pos], dv - lo, mask=m)
                return cnt + jnp.sum(mi)

            cnt = lax.fori_loop(0, BE // 16, scan, 0)
            allm = lanes < 16
            for pad in range(2):
                ppos = cnt + pad * 16 + lanes
                plsc.store_scatter(lsrc, [ppos], lanes * 0 + N, mask=allm)
                plsc.store_scatter(loff, [ppos], lanes * 0 + RNG, mask=allm)

            ng2 = (cnt + 31) // 32
            fire(0, hr0, sh0)

            def lp(g2, c2):
                g = g2 * 2
                waitg(hr0, sh0)
                fire(g + 1, hr1, sh1)
                scat(g, hr0)
                waitg(hr1, sh1)
                fire(g + 2, hr0, sh0)
                scat(g + 1, hr1)
                return c2

            lax.fori_loop(0, ng2, lp, 0)
            waitg(hr0, sh0)
            return carry

        lax.fori_loop(0, NBLK, blk_body, 0)
        plsc.subcore_barrier()

        rows_lo = lo + s * RT
        nch = jnp.clip((NP - rows_lo) // 16, 0, RT // 16)

        def wb(t, carry):
            r = rows_lo + t * 16
            pltpu.sync_copy(sh_acc.at[pl.ds(s * RT + t * 16, 16)],
                            out_h.at[pl.ds(r, 16)])
            return carry

        lax.fori_loop(0, nch, wb, 0)
        plsc.subcore_barrier()


def _scdma(table, srcp, dstp, z784):
    mesh = plsc.VectorSubcoreMesh(core_axis_name="c", subcore_axis_name="s")
    f = pl.kernel(
        _scdma_body, mesh=mesh,
        out_type=jax.ShapeDtypeStruct((NP, F), jnp.float32),
        scratch_types=[
            pltpu.VMEM_SHARED((RNG + 8, F), jnp.float32),
            pltpu.VMEM((BE,), jnp.int32),
            pltpu.VMEM((BE,), jnp.int32),
            pltpu.VMEM((LCAP,), jnp.int32),
            pltpu.VMEM((LCAP,), jnp.int32),
            pltpu.VMEM((16, F), jnp.float32),
            pltpu.VMEM((16, F), jnp.float32),
            pltpu.SemaphoreType.DMA,
            pltpu.SemaphoreType.DMA,
        ],
        compiler_params=_SC_PARAMS)
    return f(table, srcp, dstp, z784)


def kernel(x, seq_embed, W_gat, att_src, att_dst, b_gat, W_gcn, b_gcn,
           W_fcg1, b_fcg1, W_fcg2, b_fcg2, Wc1, bc1, Wc2, bc2, Wc3, bc3,
           Wc4, bc4, W_fcxt, b_fcxt, W_fc1, b_fc1, W_fc2, b_fc2,
           W_out, b_out, edge_index, batch):
    f32 = jnp.float32

    x_p = jnp.zeros((NP, 128), f32).at[:N, :D].set(x)
    w_gat_p = jnp.zeros((128, F), f32).at[:D, :H * D].set(W_gat)
    rows = jnp.arange(H * D)
    a_s = jnp.zeros((F, 16), f32).at[rows, rows // D].set(att_src.reshape(-1))
    a_d = jnp.zeros((F, 16), f32).at[rows, rows // D].set(att_dst.reshape(-1))
    b_gat_p = jnp.zeros((1, F), f32).at[0, :H * D].set(b_gat)
    w_gcn_p = jnp.zeros((F, F), f32).at[:H * D, :H * D].set(W_gcn)
    b_gcn_p = jnp.zeros((1, F), f32).at[0, :H * D].set(b_gcn)

    i32 = jnp.int32
    srcp = jnp.full((EP,), N, i32).at[:E + N].set(
        jnp.concatenate([edge_index[0], jnp.arange(N, dtype=i32)]))
    dstp = jnp.full((EP,), N, i32).at[:E + N].set(
        jnp.concatenate([edge_index[1], jnp.arange(N, dtype=i32)]))

    h, as_q, ad_q = _k1(x_p, w_gat_p, a_s, a_d)

    z0 = jnp.zeros((NP, 16), f32)
    z784 = jnp.zeros((16, F), f32)
    hm_gat = jnp.minimum(jnp.arange(F, dtype=i32) // D, 9).reshape(49, 16)
    alpha, zq = _sca(as_q, ad_q, srcp, dstp, z0)
    gat_out = _scagg(h, alpha, srcp, dstp, hm_gat, z784)

    h2 = _k2(gat_out, b_gat_p, w_gcn_p, zq)

    gcn_out = _scdma(h2, srcp, dstp, z784)

    onehot = jnp.zeros((NP, 8), f32).at[jnp.arange(N), batch].set(1.0)
    gm, gs = _k3(gcn_out, b_gcn_p, onehot, zq)
    cnt = jnp.maximum(jnp.sum(onehot, axis=0), 1.0)
    ga = gs / cnt[:, None]
    G1 = 1536
    w1a = jnp.zeros((F, G1), f32).at[:H * D, :1500].set(W_fcg1[:H * D])
    w1b = jnp.zeros((F, G1), f32).at[:H * D, :1500].set(W_fcg1[H * D:])
    b1 = jnp.zeros((1, G1), f32).at[0, :1500].set(b_fcg1)
    w2 = jnp.zeros((G1, 128), f32).at[:1500].set(W_fcg2)
    gfeat = _k4(gm, ga, w1a, w1b, b1, w2, b_fcg2.reshape(1, 128))

    xs_p = jnp.zeros((B, 1032, SD), f32).at[:, :L, :].set(seq_embed)
    y1 = _conv_stage(xs_p, jnp.transpose(Wc1, (2, 1, 0)), bc1, 5, 4, 256, SD, 256, True)
    x2 = jnp.zeros((B, 520, 512), f32).at[:, :512, :].set(y1.reshape(B, 512, 512))
    y2 = _conv_stage(x2, jnp.transpose(Wc2, (2, 1, 0)), bc2, 5, 2, 256, 256, SD, False)
    x3 = jnp.zeros((B, 264, 2048), f32).at[:, :256, :].set(y2.reshape(B, 256, 2048))
    y3 = _conv_stage(x3, jnp.transpose(Wc3, (2, 1, 0)), bc3, 5, 1, 256, SD, 256, False)
    x4 = jnp.zeros((B, 136, 512), f32).at[:, :128, :].set(y3.reshape(B, 128, 512))
    y4 = _conv_stage(x4, jnp.transpose(Wc4, (2, 1, 0)), bc4, 3, 1, 128, 256, 128, False)
    pooled4 = _s5(y4.reshape(B, 64, 256))
    xt_flat = jnp.zeros((8, 8192), f32).at[:B].set(pooled4.reshape(B, 8192))

    lidx = jnp.arange(8192) // 128
    cidx = jnp.arange(8192) % 128
    srcrow = jnp.where(lidx < 61, cidx * 61 + lidx, 0)
    wp = jnp.where((lidx < 61)[:, None], W_fcxt[srcrow], 0.0)
    gf = jnp.zeros((8, 128), f32).at[:B].set(gfeat[:B])
    w1a_f = W_fc1[:128]
    w1b_f = W_fc1[128:]
    wo = jnp.zeros((256, 128), f32).at[:, 0].set(W_out[:, 0])
    bo = jnp.zeros((1, 128), f32).at[0, 0].set(b_out[0])
    out = _s6(xt_flat, wp, b_fcxt.reshape(1, 1024), gf, w1a_f, w1b_f,
              b_fc1.reshape(1, 1024), W_fc2, b_fc2.reshape(1, 256),
              wo, bo)
    return out[:B, :1]
